# baseline (speedup 1.0000x reference)
import jax
import jax.numpy as jnp
from jax.experimental import pallas as pl
from jax.experimental.pallas import tpu as pltpu


_R = 256


def _vq_body(a_ref, c_ref, bins_ref, res_ref, cnt_ref, chi_ref, clo_ref):
    K = c_ref.shape[0]
    R = a_ref.shape[1]

    @pl.when(pl.program_id(0) + pl.program_id(1) == 0)
    def _precompute():
        c = c_ref[...]
        chi = c.astype(jnp.bfloat16)
        chi_ref[...] = chi
        clo_ref[...] = (c - chi.astype(jnp.float32)).astype(jnp.bfloat16)
        cn = jnp.sum(c * c, axis=1)
        cnt_ref[...] = cn[None, :]

    a = a_ref[...].reshape(R, a_ref.shape[2])
    c = c_ref[...]
    dots = jax.lax.dot_general(
        a, c, (((1,), (1,)), ((), ())),
        preferred_element_type=jnp.float32)
    score = (-2.0) * dots + cnt_ref[...]
    m = jnp.min(score, axis=1, keepdims=True)
    eq = score == m
    k_iota = jax.lax.broadcasted_iota(
        jnp.int32, score.shape, 1).astype(jnp.float32)
    idx = jnp.min(jnp.where(eq, k_iota, float(2 * K)), axis=1)
    bins_ref[...] = idx.astype(jnp.int32)[None, :, None]
    onehot = eq.astype(jnp.bfloat16)
    center = jax.lax.dot_general(
        onehot, chi_ref[...], (((1,), (0,)), ((), ())),
        preferred_element_type=jnp.float32)
    center_lo = jax.lax.dot_general(
        onehot, clo_ref[...], (((1,), (0,)), ((), ())),
        preferred_element_type=jnp.float32)
    res_ref[...] = (a - (center + center_lo))[None]


def kernel(action, centroids):
    B, T, D = action.shape
    K = centroids.shape[0]
    grid = (B, T // _R)
    bins, res = pl.pallas_call(
        _vq_body,
        grid=grid,
        in_specs=[
            pl.BlockSpec((1, _R, D), lambda b, t: (b, t, 0)),
            pl.BlockSpec((K, D), lambda b, t: (0, 0)),
        ],
        out_specs=[
            pl.BlockSpec((1, _R, 1), lambda b, t: (b, t, 0)),
            pl.BlockSpec((1, _R, D), lambda b, t: (b, t, 0)),
        ],
        out_shape=[
            jax.ShapeDtypeStruct((B, T, 1), jnp.int32),
            jax.ShapeDtypeStruct((B, T, D), jnp.float32),
        ],
        scratch_shapes=[
            pltpu.VMEM((1, K), jnp.float32),
            pltpu.VMEM((K, D), jnp.bfloat16),
            pltpu.VMEM((K, D), jnp.bfloat16),
        ],
    )(action, centroids)
    return bins.astype(jnp.int64), res


# speedup vs baseline: 1.0291x; 1.0291x over previous
import jax
import jax.numpy as jnp
from jax.experimental import pallas as pl
from jax.experimental.pallas import tpu as pltpu


_R = 512


def _vq_body(a_ref, c_ref, bins_ref, res_ref, cnt_ref, chi_ref, clo_ref):
    K = c_ref.shape[0]
    R = a_ref.shape[1]

    @pl.when(pl.program_id(0) + pl.program_id(1) == 0)
    def _precompute():
        c = c_ref[...]
        chi = c.astype(jnp.bfloat16)
        chi_ref[...] = chi
        clo_ref[...] = (c - chi.astype(jnp.float32)).astype(jnp.bfloat16)
        cn = jnp.sum(c * c, axis=1)
        cnt_ref[...] = -cn[None, :]

    a = a_ref[...].reshape(R, a_ref.shape[2])
    c = c_ref[...]
    dots = jax.lax.dot_general(
        a, c, (((1,), (1,)), ((), ())),
        preferred_element_type=jnp.float32)
    score = 2.0 * dots + cnt_ref[...]
    m = jnp.max(score, axis=1, keepdims=True)
    eq = score == m
    rev_iota = float(K - 1) - jax.lax.broadcasted_iota(
        jnp.int32, score.shape, 1).astype(jnp.float32)
    idx_rev = jnp.max(jnp.where(eq, rev_iota, -1.0), axis=1)
    bins_ref[...] = (
        (K - 1) - idx_rev.astype(jnp.int32))[None, :, None]
    onehot = eq.astype(jnp.bfloat16)
    center = jax.lax.dot_general(
        onehot, chi_ref[...], (((1,), (0,)), ((), ())),
        preferred_element_type=jnp.float32)
    center_lo = jax.lax.dot_general(
        onehot, clo_ref[...], (((1,), (0,)), ((), ())),
        preferred_element_type=jnp.float32)
    res_ref[...] = (a - (center + center_lo))[None]


def kernel(action, centroids):
    B, T, D = action.shape
    K = centroids.shape[0]
    grid = (B, T // _R)
    bins, res = pl.pallas_call(
        _vq_body,
        grid=grid,
        in_specs=[
            pl.BlockSpec((1, _R, D), lambda b, t: (b, t, 0)),
            pl.BlockSpec((K, D), lambda b, t: (0, 0)),
        ],
        out_specs=[
            pl.BlockSpec((1, _R, 1), lambda b, t: (b, t, 0)),
            pl.BlockSpec((1, _R, D), lambda b, t: (b, t, 0)),
        ],
        out_shape=[
            jax.ShapeDtypeStruct((B, T, 1), jnp.int32),
            jax.ShapeDtypeStruct((B, T, D), jnp.float32),
        ],
        scratch_shapes=[
            pltpu.VMEM((1, K), jnp.float32),
            pltpu.VMEM((K, D), jnp.bfloat16),
            pltpu.VMEM((K, D), jnp.bfloat16),
        ],
    )(action, centroids)
    return bins.astype(jnp.int64), res


# speedup vs baseline: 1.0956x; 1.0646x over previous
import jax
import jax.numpy as jnp
from jax.experimental import pallas as pl
from jax.experimental.pallas import tpu as pltpu


_R = 1024


def _vq_body(a_ref, c_ref, bins_ref, res_ref,
             cnt_ref, c2_ref, chi_ref, clo_ref, riota_ref):
    K = c_ref.shape[0]
    R = a_ref.shape[1]

    @pl.when(pl.program_id(0) + pl.program_id(1) == 0)
    def _precompute():
        c = c_ref[...]
        chi = c.astype(jnp.bfloat16)
        chi_ref[...] = chi
        clo_ref[...] = (c - chi.astype(jnp.float32)).astype(jnp.bfloat16)
        c2_ref[...] = c + c
        cn = jnp.sum(c * c, axis=1)
        cnt_ref[...] = -cn[None, :]
        riota_ref[...] = float(K - 1) - jax.lax.broadcasted_iota(
            jnp.int32, (1, K), 1).astype(jnp.float32)

    a = a_ref[...].reshape(R, a_ref.shape[2])
    dots2 = jax.lax.dot_general(
        a, c2_ref[...], (((1,), (1,)), ((), ())),
        preferred_element_type=jnp.float32)
    score = dots2 + cnt_ref[...]
    m = jnp.max(score, axis=1, keepdims=True)
    eq = score == m
    idx_rev = jnp.max(jnp.where(eq, riota_ref[...], -1.0),
                      axis=1, keepdims=True)
    bins_ref[...] = (
        (K - 1) - idx_rev.astype(jnp.int32))[None]
    onehot = eq.astype(jnp.bfloat16)
    center = jax.lax.dot_general(
        onehot, chi_ref[...], (((1,), (0,)), ((), ())),
        preferred_element_type=jnp.float32)
    center_lo = jax.lax.dot_general(
        onehot, clo_ref[...], (((1,), (0,)), ((), ())),
        preferred_element_type=jnp.float32)
    res_ref[...] = (a - (center + center_lo))[None]


def kernel(action, centroids):
    B, T, D = action.shape
    K = centroids.shape[0]
    grid = (B, T // _R)
    bins, res = pl.pallas_call(
        _vq_body,
        grid=grid,
        in_specs=[
            pl.BlockSpec((1, _R, D), lambda b, t: (b, t, 0)),
            pl.BlockSpec((K, D), lambda b, t: (0, 0)),
        ],
        out_specs=[
            pl.BlockSpec((1, _R, 1), lambda b, t: (b, t, 0)),
            pl.BlockSpec((1, _R, D), lambda b, t: (b, t, 0)),
        ],
        out_shape=[
            jax.ShapeDtypeStruct((B, T, 1), jnp.int32),
            jax.ShapeDtypeStruct((B, T, D), jnp.float32),
        ],
        scratch_shapes=[
            pltpu.VMEM((1, K), jnp.float32),
            pltpu.VMEM((K, D), jnp.float32),
            pltpu.VMEM((K, D), jnp.bfloat16),
            pltpu.VMEM((K, D), jnp.bfloat16),
            pltpu.VMEM((1, K), jnp.float32),
        ],
    )(action, centroids)
    return bins.astype(jnp.int64), res


# speedup vs baseline: 1.2437x; 1.1351x over previous
import jax
import jax.numpy as jnp
from jax.experimental import pallas as pl
from jax.experimental.pallas import tpu as pltpu


_R = 1024


def _vq_body(a_ref, c_ref, bins_ref, res_ref,
             cnt_ref, c2_ref, cat_ref, riota_ref):
    K = c_ref.shape[0]
    R = a_ref.shape[1]
    D = a_ref.shape[2]

    @pl.when(pl.program_id(0) + pl.program_id(1) == 0)
    def _precompute():
        c = c_ref[...]
        chi = c.astype(jnp.bfloat16)
        cat_ref[:, :D] = chi
        cat_ref[:, D:] = (c - chi.astype(jnp.float32)).astype(jnp.bfloat16)
        c2_ref[...] = c + c
        cn = jnp.sum(c * c, axis=1)
        cnt_ref[...] = -cn[None, :]
        riota_ref[...] = float(K - 1) - jax.lax.broadcasted_iota(
            jnp.int32, (1, K), 1).astype(jnp.float32)

    a = a_ref[...].reshape(R, a_ref.shape[2])
    dots2 = jax.lax.dot_general(
        a, c2_ref[...], (((1,), (1,)), ((), ())),
        preferred_element_type=jnp.float32)
    score = dots2 + cnt_ref[...]
    m = jnp.max(score, axis=1, keepdims=True)
    eq = score == m
    idx_rev = jnp.max(jnp.where(eq, riota_ref[...], -1.0),
                      axis=1, keepdims=True)
    bins_ref[...] = (
        (K - 1) - idx_rev.astype(jnp.int32))[None]
    onehot = eq.astype(jnp.bfloat16)
    center2 = jax.lax.dot_general(
        onehot, cat_ref[...], (((1,), (0,)), ((), ())),
        preferred_element_type=jnp.float32)
    res_ref[...] = (a - (center2[:, :D] + center2[:, D:]))[None]


def kernel(action, centroids):
    B, T, D = action.shape
    K = centroids.shape[0]
    grid = (B, T // _R)
    bins, res = pl.pallas_call(
        _vq_body,
        grid=grid,
        in_specs=[
            pl.BlockSpec((1, _R, D), lambda b, t: (b, t, 0)),
            pl.BlockSpec((K, D), lambda b, t: (0, 0)),
        ],
        out_specs=[
            pl.BlockSpec((1, _R, 1), lambda b, t: (b, t, 0)),
            pl.BlockSpec((1, _R, D), lambda b, t: (b, t, 0)),
        ],
        out_shape=[
            jax.ShapeDtypeStruct((B, T, 1), jnp.int32),
            jax.ShapeDtypeStruct((B, T, D), jnp.float32),
        ],
        scratch_shapes=[
            pltpu.VMEM((1, K), jnp.float32),
            pltpu.VMEM((K, D), jnp.float32),
            pltpu.VMEM((K, 2 * D), jnp.bfloat16),
            pltpu.VMEM((1, K), jnp.float32),
        ],
    )(action, centroids)
    return bins.astype(jnp.int64), res


# speedup vs baseline: 1.4109x; 1.1345x over previous
import jax
import jax.numpy as jnp
from jax.experimental import pallas as pl
from jax.experimental.pallas import tpu as pltpu


_R = 1024


def _vq_body(a_ref, c_ref, bins_ref, res_ref,
             cnt_ref, c2_ref, cat_ref, riota_ref):
    K = c_ref.shape[0]
    R = a_ref.shape[1]
    D = a_ref.shape[2]

    @pl.when(pl.program_id(0) + pl.program_id(1) == 0)
    def _precompute():
        c = c_ref[...]
        chi = c.astype(jnp.bfloat16)
        cat_ref[:, :D] = chi
        cat_ref[:, D:] = (c - chi.astype(jnp.float32)).astype(jnp.bfloat16)
        c2_ref[...] = c + c
        cn = jnp.sum(c * c, axis=1)
        cnt_ref[...] = -cn[None, :]
        riota_ref[...] = float(K - 1) - jax.lax.broadcasted_iota(
            jnp.int32, (1, K), 1).astype(jnp.float32)

    a = a_ref[...].reshape(R, a_ref.shape[2])
    _NCHUNK = 4
    H = R // _NCHUNK
    for h in range(_NCHUNK):
        rows = slice(h * H, (h + 1) * H)
        ah = a[rows, :]
        dots2 = jax.lax.dot_general(
            ah, c2_ref[...], (((1,), (1,)), ((), ())),
            preferred_element_type=jnp.float32)
        score = dots2 + cnt_ref[...]
        m = jnp.max(score, axis=1, keepdims=True)
        eq = score == m
        idx_rev = jnp.max(jnp.where(eq, riota_ref[...], -1.0),
                          axis=1, keepdims=True)
        bins_ref[0, rows, :] = (
            (K - 1) - idx_rev.astype(jnp.int32))
        onehot = eq.astype(jnp.bfloat16)
        center2 = jax.lax.dot_general(
            onehot, cat_ref[...], (((1,), (0,)), ((), ())),
            preferred_element_type=jnp.float32)
        res_ref[0, rows, :] = ah - (center2[:, :D] + center2[:, D:])


def kernel(action, centroids):
    B, T, D = action.shape
    K = centroids.shape[0]
    grid = (B, T // _R)
    bins, res = pl.pallas_call(
        _vq_body,
        grid=grid,
        in_specs=[
            pl.BlockSpec((1, _R, D), lambda b, t: (b, t, 0)),
            pl.BlockSpec((K, D), lambda b, t: (0, 0)),
        ],
        out_specs=[
            pl.BlockSpec((1, _R, 1), lambda b, t: (b, t, 0)),
            pl.BlockSpec((1, _R, D), lambda b, t: (b, t, 0)),
        ],
        out_shape=[
            jax.ShapeDtypeStruct((B, T, 1), jnp.int32),
            jax.ShapeDtypeStruct((B, T, D), jnp.float32),
        ],
        scratch_shapes=[
            pltpu.VMEM((1, K), jnp.float32),
            pltpu.VMEM((K, D), jnp.float32),
            pltpu.VMEM((K, 2 * D), jnp.bfloat16),
            pltpu.VMEM((1, K), jnp.float32),
        ],
    )(action, centroids)
    return bins.astype(jnp.int64), res


# speedup vs baseline: 1.6349x; 1.1588x over previous
import jax
import jax.numpy as jnp
from jax.experimental import pallas as pl
from jax.experimental.pallas import tpu as pltpu


_R = 2048


def _vq_body(a_ref, c_ref, bins_ref, res_ref,
             cnt_ref, c2_ref, cat_ref, riota_ref):
    K = c_ref.shape[0]
    R = a_ref.shape[1]
    D = a_ref.shape[2]

    @pl.when(pl.program_id(0) + pl.program_id(1) == 0)
    def _precompute():
        c = c_ref[...]
        chi = c.astype(jnp.bfloat16)
        cat_ref[:, :D] = chi
        cat_ref[:, D:] = (c - chi.astype(jnp.float32)).astype(jnp.bfloat16)
        c2_ref[...] = c + c
        cn = jnp.sum(c * c, axis=1)
        cnt_ref[...] = -cn[None, :]
        riota_ref[...] = float(K - 1) - jax.lax.broadcasted_iota(
            jnp.int32, (1, K), 1).astype(jnp.float32)

    a = a_ref[...].reshape(R, a_ref.shape[2])
    _NCHUNK = 4
    H = R // _NCHUNK
    for h in range(_NCHUNK):
        rows = slice(h * H, (h + 1) * H)
        ah = a[rows, :]
        dots2 = jax.lax.dot_general(
            ah, c2_ref[...], (((1,), (1,)), ((), ())),
            preferred_element_type=jnp.float32)
        score = dots2 + cnt_ref[...]
        m = jnp.max(score, axis=1, keepdims=True)
        eq = score == m
        idx_rev = jnp.max(jnp.where(eq, riota_ref[...], -1.0),
                          axis=1, keepdims=True)
        bins_ref[0, rows, :] = (
            (K - 1) - idx_rev.astype(jnp.int32))
        onehot = eq.astype(jnp.bfloat16)
        center2 = jax.lax.dot_general(
            onehot, cat_ref[...], (((1,), (0,)), ((), ())),
            preferred_element_type=jnp.float32)
        res_ref[0, rows, :] = ah - (center2[:, :D] + center2[:, D:])


def kernel(action, centroids):
    B, T, D = action.shape
    K = centroids.shape[0]
    grid = (B, T // _R)
    bins, res = pl.pallas_call(
        _vq_body,
        grid=grid,
        in_specs=[
            pl.BlockSpec((1, _R, D), lambda b, t: (b, t, 0)),
            pl.BlockSpec((K, D), lambda b, t: (0, 0)),
        ],
        out_specs=[
            pl.BlockSpec((1, _R, 1), lambda b, t: (b, t, 0)),
            pl.BlockSpec((1, _R, D), lambda b, t: (b, t, 0)),
        ],
        out_shape=[
            jax.ShapeDtypeStruct((B, T, 1), jnp.int32),
            jax.ShapeDtypeStruct((B, T, D), jnp.float32),
        ],
        scratch_shapes=[
            pltpu.VMEM((1, K), jnp.float32),
            pltpu.VMEM((K, D), jnp.float32),
            pltpu.VMEM((K, 2 * D), jnp.bfloat16),
            pltpu.VMEM((1, K), jnp.float32),
        ],
    )(action, centroids)
    return bins.astype(jnp.int64), res


# speedup vs baseline: 1.6971x; 1.0380x over previous
import jax
import jax.numpy as jnp
from jax.experimental import pallas as pl
from jax.experimental.pallas import tpu as pltpu


_NB = 2
_H = 512


def _vq_body(a_ref, c_ref, bins_ref, res_ref,
             cnt_ref, c2_ref, cat_ref, riota_ref):
    K = c_ref.shape[0]
    NB, T, D = a_ref.shape

    @pl.when(pl.program_id(0) == 0)
    def _precompute():
        c = c_ref[...]
        chi = c.astype(jnp.bfloat16)
        cat_ref[:, :D] = chi
        cat_ref[:, D:] = (c - chi.astype(jnp.float32)).astype(jnp.bfloat16)
        c2_ref[...] = c + c
        cn = jnp.sum(c * c, axis=1)
        cnt_ref[...] = -cn[None, :]
        riota_ref[...] = float(K - 1) - jax.lax.broadcasted_iota(
            jnp.int32, (1, K), 1).astype(jnp.float32)

    a = a_ref[...].reshape(NB * T, D)
    for h in range(NB * T // _H):
        rows = slice(h * _H, (h + 1) * _H)
        b_i, t0 = (h * _H) // T, (h * _H) % T
        trows = slice(t0, t0 + _H)
        ah = a[rows, :]
        dots2 = jax.lax.dot_general(
            ah, c2_ref[...], (((1,), (1,)), ((), ())),
            preferred_element_type=jnp.float32)
        score = dots2 + cnt_ref[...]
        m = jnp.max(score, axis=1, keepdims=True)
        eq = score == m
        idx_rev = jnp.max(jnp.where(eq, riota_ref[...], -1.0),
                          axis=1, keepdims=True)
        bins_ref[b_i, trows, :] = (
            (K - 1) - idx_rev.astype(jnp.int32))
        onehot = eq.astype(jnp.bfloat16)
        center2 = jax.lax.dot_general(
            onehot, cat_ref[...], (((1,), (0,)), ((), ())),
            preferred_element_type=jnp.float32)
        res_ref[b_i, trows, :] = ah - (center2[:, :D] + center2[:, D:])


def kernel(action, centroids):
    B, T, D = action.shape
    K = centroids.shape[0]
    grid = (B // _NB,)
    bins, res = pl.pallas_call(
        _vq_body,
        grid=grid,
        in_specs=[
            pl.BlockSpec((_NB, T, D), lambda i: (i, 0, 0)),
            pl.BlockSpec((K, D), lambda i: (0, 0)),
        ],
        out_specs=[
            pl.BlockSpec((_NB, T, 1), lambda i: (i, 0, 0)),
            pl.BlockSpec((_NB, T, D), lambda i: (i, 0, 0)),
        ],
        out_shape=[
            jax.ShapeDtypeStruct((B, T, 1), jnp.int32),
            jax.ShapeDtypeStruct((B, T, D), jnp.float32),
        ],
        scratch_shapes=[
            pltpu.VMEM((1, K), jnp.float32),
            pltpu.VMEM((K, D), jnp.float32),
            pltpu.VMEM((K, 2 * D), jnp.bfloat16),
            pltpu.VMEM((1, K), jnp.float32),
        ],
    )(action, centroids)
    return bins.astype(jnp.int64), res


# speedup vs baseline: 1.7101x; 1.0077x over previous
import jax
import jax.numpy as jnp
from jax.experimental import pallas as pl
from jax.experimental.pallas import tpu as pltpu


_NB = 4
_H = 512


def _vq_body(a_ref, c_ref, bins_ref, res_ref,
             cnt_ref, c2_ref, cat_ref, riota_ref):
    K = c_ref.shape[0]
    NB, T, D = a_ref.shape

    @pl.when(pl.program_id(0) == 0)
    def _precompute():
        c = c_ref[...]
        chi = c.astype(jnp.bfloat16)
        cat_ref[:, :D] = chi
        cat_ref[:, D:] = (c - chi.astype(jnp.float32)).astype(jnp.bfloat16)
        c2_ref[...] = c + c
        cn = jnp.sum(c * c, axis=1)
        cnt_ref[...] = -cn[None, :]
        riota_ref[...] = float(K - 1) - jax.lax.broadcasted_iota(
            jnp.int32, (1, K), 1).astype(jnp.float32)

    a = a_ref[...].reshape(NB * T, D)
    for h in range(NB * T // _H):
        rows = slice(h * _H, (h + 1) * _H)
        b_i, t0 = (h * _H) // T, (h * _H) % T
        trows = slice(t0, t0 + _H)
        ah = a[rows, :]
        dots2 = jax.lax.dot_general(
            ah, c2_ref[...], (((1,), (1,)), ((), ())),
            preferred_element_type=jnp.float32)
        score = dots2 + cnt_ref[...]
        m = jnp.max(score, axis=1, keepdims=True)
        eq = score == m
        idx_rev = jnp.max(jnp.where(eq, riota_ref[...], -1.0),
                          axis=1, keepdims=True)
        bins_ref[b_i, trows, :] = (
            (K - 1) - idx_rev.astype(jnp.int32))
        onehot = eq.astype(jnp.bfloat16)
        center2 = jax.lax.dot_general(
            onehot, cat_ref[...], (((1,), (0,)), ((), ())),
            preferred_element_type=jnp.float32)
        res_ref[b_i, trows, :] = ah - (center2[:, :D] + center2[:, D:])


def kernel(action, centroids):
    B, T, D = action.shape
    K = centroids.shape[0]
    grid = (B // _NB,)
    bins, res = pl.pallas_call(
        _vq_body,
        grid=grid,
        in_specs=[
            pl.BlockSpec((_NB, T, D), lambda i: (i, 0, 0)),
            pl.BlockSpec((K, D), lambda i: (0, 0)),
        ],
        out_specs=[
            pl.BlockSpec((_NB, T, 1), lambda i: (i, 0, 0)),
            pl.BlockSpec((_NB, T, D), lambda i: (i, 0, 0)),
        ],
        out_shape=[
            jax.ShapeDtypeStruct((B, T, 1), jnp.int32),
            jax.ShapeDtypeStruct((B, T, D), jnp.float32),
        ],
        scratch_shapes=[
            pltpu.VMEM((1, K), jnp.float32),
            pltpu.VMEM((K, D), jnp.float32),
            pltpu.VMEM((K, 2 * D), jnp.bfloat16),
            pltpu.VMEM((1, K), jnp.float32),
        ],
    )(action, centroids)
    return bins.astype(jnp.int64), res
